# baseline (speedup 1.0000x reference)
import functools

import jax
import jax.numpy as jnp
from jax import lax
from jax.experimental import pallas as pl
from jax.experimental.pallas import tpu as pltpu
from jax.experimental.pallas import tpu_sc as plsc

SCENE = (64, 64, 8)
C = 128
NH = 8
NL = 3
NP = 4
DH = C // NH
LEVEL_SHAPES = [(64, 64), (32, 32), (16, 16)]

BQ = 256


def _pre_body(q_ref, qpos_ref, so_w_ref, so_b_ref, aw_w_ref, aw_b_ref,
              off_ref, awl_ref):
    qq = q_ref[...] + qpos_ref[...]
    off_ref[...] = jnp.dot(qq, so_w_ref[...],
                           preferred_element_type=jnp.float32) + so_b_ref[...]
    awl_ref[...] = jnp.dot(qq, aw_w_ref[...],
                           preferred_element_type=jnp.float32) + aw_b_ref[...]


def _vproj_body(val_ref, w_ref, b_ref, out_ref):
    out_ref[...] = jnp.dot(val_ref[...], w_ref[...],
                           preferred_element_type=jnp.float32) + b_ref[...]


def _post_body(s0, s1, s2, s3, s4, s5, s6, s7, q_ref, mk_ref,
               op_w_ref, op_b_ref, n1g_ref, n1b_ref,
               f1w_ref, f1b_ref, f2w_ref, f2b_ref, n2g_ref, n2b_ref, out_ref):
    s = jnp.concatenate([r[...] for r in (s0, s1, s2, s3, s4, s5, s6, s7)], axis=1)
    x = jnp.dot(s, op_w_ref[...],
                preferred_element_type=jnp.float32) + op_b_ref[...] + q_ref[...]
    m = jnp.mean(x, -1, keepdims=True)
    v = jnp.mean((x - m) * (x - m), -1, keepdims=True)
    x = (x - m) * jax.lax.rsqrt(v + 1e-5) * n1g_ref[...] + n1b_ref[...]
    h = jnp.maximum(jnp.dot(x, f1w_ref[...],
                            preferred_element_type=jnp.float32) + f1b_ref[...], 0.0)
    y = x + jnp.dot(h, f2w_ref[...],
                    preferred_element_type=jnp.float32) + f2b_ref[...]
    m2 = jnp.mean(y, -1, keepdims=True)
    v2 = jnp.mean((y - m2) * (y - m2), -1, keepdims=True)
    pts = (y - m2) * jax.lax.rsqrt(v2 + 1e-5) * n2g_ref[...] + n2b_ref[...]
    out_ref[...] = jnp.where(mk_ref[...] > 0, pts, q_ref[...])


NC_SC = 2
NS_SC = 16
NW_SC = NC_SC * NS_SC
BQS = 64
NV_TOT = 64 * 64 + 32 * 32 + 16 * 16
LEVEL_STARTS = (0, 4096, 5120)


def _sc_sample_group(q0, tab_v, off_v, awl_v, refp_v, out_v, h):
    qloc = q0 + lax.broadcasted_iota(jnp.int32, (16,), 0)

    def col(c):
        return jnp.full((16,), c, jnp.int32)

    logits = [plsc.load_gather(awl_v, [qloc, col(h * 12 + lp)]) for lp in range(12)]
    m = logits[0]
    for t in logits[1:]:
        m = jnp.maximum(m, t)
    es = [jnp.exp(t - m) for t in logits]
    s = es[0]
    for t in es[1:]:
        s = s + t
    r = 1.0 / s
    aws = [e * r for e in es]

    accs = [jnp.zeros((16,), jnp.float32) for _ in range(DH)]
    for l, (Hl, Wl) in enumerate(LEVEL_SHAPES):
        rx = plsc.load_gather(refp_v, [qloc, col(2 * l)])
        ry = plsc.load_gather(refp_v, [qloc, col(2 * l + 1)])
        for p in range(NP):
            ocol = h * 24 + l * 8 + p * 2
            x = rx + plsc.load_gather(off_v, [qloc, col(ocol)])
            y = ry + plsc.load_gather(off_v, [qloc, col(ocol + 1)])
            tx = x.astype(jnp.int32).astype(jnp.float32)
            x0 = jnp.where(tx > x, tx - 1.0, tx)
            ty = y.astype(jnp.int32).astype(jnp.float32)
            y0 = jnp.where(ty > y, ty - 1.0, ty)
            fx = x - x0
            fy = y - y0
            aw = aws[l * NP + p]
            for dx in (0, 1):
                for dy in (0, 1):
                    xi = x0 + dx
                    yi = y0 + dy
                    valid = ((xi >= 0.0) & (xi <= Wl - 1.0)
                             & (yi >= 0.0) & (yi <= Hl - 1.0))
                    wc = (fx if dx else 1.0 - fx) * (fy if dy else 1.0 - fy)
                    wt = jnp.where(valid, wc * aw, 0.0)
                    xc = jnp.clip(xi, 0.0, Wl - 1.0).astype(jnp.int32)
                    yc = jnp.clip(yi, 0.0, Hl - 1.0).astype(jnp.int32)
                    row = (yc * Wl + xc + LEVEL_STARTS[l]) * DH
                    for d in range(DH):
                        g = plsc.load_gather(tab_v, [row + d])
                        accs[d] = accs[d] + wt * g
    for d in range(DH):
        plsc.store_scatter(out_v, [qloc, col(d)], accs[d])


NSEG = 16
SEGQ = SCENE[0] * SCENE[1] * SCENE[2] // NSEG


def _sc_compact_body(vol_hbm, idxc_hbm, cnt_hbm, mask_hbm,
                     loc_v, vol_v, win_v, tmp_v, idxl_v, mk_v, cnt16_v, table_s):
    X, Y, Z = SCENE
    N = X * Y * Z
    t = lax.axis_index("s")
    lane = lax.broadcasted_iota(jnp.int32, (16,), 0)

    def zbody(i, _):
        loc_v[pl.ds(i * 16, 16)] = jnp.zeros((16,), jnp.int32)
        return 0

    lax.fori_loop(0, N // 16, zbody, 0)
    pltpu.sync_copy(vol_hbm.at[pl.ds(t * (SEGQ * 3), SEGQ * 3)], vol_v)

    def sbody(g, _):
        r3 = (g * 16 + lane) * 3
        gx = plsc.load_gather(vol_v, [r3])
        gy = plsc.load_gather(vol_v, [r3 + 1])
        gz = plsc.load_gather(vol_v, [r3 + 2])
        keep = ((gx >= 0) & (gx < X) & (gy >= 0) & (gy < Y)
                & (gz >= 0) & (gz < Z))
        cx = jnp.clip(gx, 0, X - 1)
        cy = jnp.clip(gy, 0, Y - 1)
        cz = jnp.clip(gz, 0, Z - 1)
        f = (cx * Y + cy) * Z + cz
        plsc.store_scatter(loc_v, [f], jnp.ones((16,), jnp.int32), mask=keep)
        return 0

    lax.fori_loop(0, SEGQ // 16, sbody, 0)
    pltpu.sync_copy(loc_v, table_s.at[t])
    plsc.subcore_barrier()

    pltpu.sync_copy(table_s.at[0, pl.ds(t * SEGQ, SEGQ)], win_v)
    for j in range(1, NSEG):
        pltpu.sync_copy(table_s.at[j, pl.ds(t * SEGQ, SEGQ)], tmp_v)

        def abody(i, _):
            win_v[pl.ds(i * 16, 16)] = (win_v[pl.ds(i * 16, 16)]
                                        + tmp_v[pl.ds(i * 16, 16)])
            return 0

        lax.fori_loop(0, SEGQ // 16, abody, 0)

    def zbody2(i, _):
        idxl_v[pl.ds(i * 16, 16)] = jnp.zeros((16,), jnp.int32)
        return 0

    lax.fori_loop(0, (SEGQ + 64) // 16, zbody2, 0)

    def cbody(g, woff):
        cnt = win_v[pl.ds(g * 16, 16)]
        m = cnt > 0
        mk_v[pl.ds(g * 16, 16)] = m.astype(jnp.int32)
        pos = plsc.cumsum(m.astype(jnp.int32))
        widx = t * SEGQ + g * 16 + lane
        plsc.store_scatter(idxl_v, [woff + pos - 1], widx, mask=m)
        return woff + jnp.max(pos, axis=0)

    woff = lax.fori_loop(0, SEGQ // 16, cbody, jnp.int32(0))

    e0 = plsc.load_gather(idxl_v, [jnp.zeros((16,), jnp.int32)])
    pbase = (woff // 16) * 16
    rem = woff - pbase
    part = idxl_v[pl.ds(pbase, 16)]
    idxl_v[pl.ds(pbase, 16)] = jnp.where(lane < rem, part, e0)
    wpad = (woff + 63) // 64 * 64

    def pbody(i, _):
        idxl_v[pl.ds(pbase + 16 + i * 16, 16)] = e0
        return 0

    lax.fori_loop(0, (wpad - pbase - 16) // 16, pbody, 0)

    pltpu.sync_copy(idxl_v.at[pl.ds(0, SEGQ)], idxc_hbm.at[pl.ds(t * SEGQ, SEGQ)])
    pltpu.sync_copy(mk_v, mask_hbm.at[pl.ds(t * SEGQ, SEGQ)])
    cnt16_v[...] = jnp.broadcast_to(wpad // BQS, (16,)).astype(jnp.int32)
    pltpu.sync_copy(cnt16_v, cnt_hbm.at[pl.ds(t * 16, 16)])


def _sc_compact(vol_flat):
    N = SCENE[0] * SCENE[1] * SCENE[2]
    mesh = plsc.VectorSubcoreMesh(core_axis_name="c", subcore_axis_name="s",
                                  num_cores=1)
    f = pl.kernel(
        _sc_compact_body,
        mesh=mesh,
        out_type=(jax.ShapeDtypeStruct((N,), jnp.int32),
                  jax.ShapeDtypeStruct((NSEG * 16,), jnp.int32),
                  jax.ShapeDtypeStruct((N,), jnp.int32)),
        scratch_types=[
            pltpu.VMEM((N,), jnp.int32),
            pltpu.VMEM((SEGQ * 3,), jnp.int32),
            pltpu.VMEM((SEGQ,), jnp.int32),
            pltpu.VMEM((SEGQ,), jnp.int32),
            pltpu.VMEM((SEGQ + 64,), jnp.int32),
            pltpu.VMEM((SEGQ,), jnp.int32),
            pltpu.VMEM((16,), jnp.int32),
            pltpu.VMEM_SHARED((NSEG, N), jnp.int32),
        ],
        compiler_params=pltpu.CompilerParams(
            needs_layout_passes=False, use_tc_tiling_on_sc=False),
    )
    return f(vol_flat)


def _sc_sample_body(vt_hbm, off_hbm, awl_hbm, refp_hbm, idxc_hbm, cnt_hbm, out_hbm,
                    tab_v, off_v, awl_v, refp_v, out_v, idx_v, idx2_v, cnt_v):
    N = SCENE[0] * SCENE[1] * SCENE[2]
    w = lax.axis_index("s") * NC_SC + lax.axis_index("c")
    h = w % NH
    chunk = w // NH
    nchunks = NW_SC // NH

    pltpu.sync_copy(vt_hbm.at[pl.ds(h * (NV_TOT * DH), NV_TOT * DH)], tab_v)
    pltpu.sync_copy(cnt_hbm, cnt_v)
    lane = lax.broadcasted_iota(jnp.int32, (16,), 0)
    cnts = plsc.load_gather(cnt_v, [lane * 16])

    def bwork(base):
        pltpu.sync_copy(idxc_hbm.at[pl.ds(base, BQS)], idx_v)
        pltpu.sync_copy(off_hbm.at[idx_v], off_v)
        pltpu.sync_copy(awl_hbm.at[idx_v], awl_v)
        pltpu.sync_copy(refp_hbm.at[idx_v], refp_v)
        for gg in range(BQS // 16):
            idx2_v[pl.ds(gg * 16, 16)] = idx_v[pl.ds(gg * 16, 16)] + h * N

        def qbody(qg, _):
            _sc_sample_group(qg * 16, tab_v, off_v, awl_v, refp_v, out_v, h)
            return 0

        lax.fori_loop(0, BQS // 16, qbody, 0)
        pltpu.sync_copy(out_v, out_hbm.at[idx2_v])

    def seg_body(s, _):
        nbs = jnp.sum(jnp.where(lane == s, cnts, 0), axis=0)
        nbc = (nbs - chunk + 3) // 4

        def bbody(bi, _):
            bwork(s * SEGQ + (bi * nchunks + chunk) * BQS)
            return 0

        lax.fori_loop(0, nbc, bbody, 0)
        return 0

    lax.fori_loop(0, NSEG, seg_body, 0)


def _sc_sample(vt, off, awl, refp, idxc, cnt):
    N = SCENE[0] * SCENE[1] * SCENE[2]
    mesh = plsc.VectorSubcoreMesh(core_axis_name="c", subcore_axis_name="s")
    f = pl.kernel(
        _sc_sample_body,
        mesh=mesh,
        out_type=jax.ShapeDtypeStruct((NH * N, DH), jnp.float32),
        scratch_types=[
            pltpu.VMEM((NV_TOT * DH,), jnp.float32),
            pltpu.VMEM((BQS, NH * NL * NP * 2), jnp.float32),
            pltpu.VMEM((BQS, NH * NL * NP), jnp.float32),
            pltpu.VMEM((BQS, 16), jnp.float32),
            pltpu.VMEM((BQS, DH), jnp.float32),
            pltpu.VMEM((BQS,), jnp.int32),
            pltpu.VMEM((BQS,), jnp.int32),
            pltpu.VMEM((NSEG * 16,), jnp.int32),
        ],
        compiler_params=pltpu.CompilerParams(
            needs_layout_passes=False, use_tc_tiling_on_sc=False),
    )
    out = f(vt.reshape(-1), off, awl, refp, idxc, cnt)
    return out.reshape(NH, N, DH)


def _bilinear_all(vh, x, y, Hl, Wl):
    x0 = jnp.floor(x)
    y0 = jnp.floor(y)

    def g(xi, yi):
        valid = (xi >= 0) & (xi <= Wl - 1) & (yi >= 0) & (yi <= Hl - 1)
        xc = jnp.clip(xi, 0, Wl - 1).astype(jnp.int32)
        yc = jnp.clip(yi, 0, Hl - 1).astype(jnp.int32)
        idx = yc * Wl + xc
        got = jnp.take_along_axis(vh, idx.reshape(NH, -1, 1), axis=1)
        return got.reshape(xi.shape + (DH,)) * valid[..., None]

    w00 = (x0 + 1 - x) * (y0 + 1 - y)
    w01 = (x - x0) * (y0 + 1 - y)
    w10 = (x0 + 1 - x) * (y - y0)
    w11 = (x - x0) * (y - y0)
    return (g(x0, y0) * w00[..., None] + g(x0 + 1, y0) * w01[..., None]
            + g(x0, y0 + 1) * w10[..., None] + g(x0 + 1, y0 + 1) * w11[..., None])


def kernel(scene_embed, feat0, feat1, feat2, scene_pos, ref_pix, vol_pts,
           vp_w, vp_b, so_w, so_b, aw_w, aw_b, op_w, op_b,
           n1_g, n1_b, f1_w, f1_b, f2_w, f2_b, n2_g, n2_b):
    X, Y, Z = SCENE
    N = X * Y * Z
    q = scene_embed[0]
    qpos = scene_pos[0]
    vol = vol_pts[0]
    ref = ref_pix[0]

    idxc, cnt, maskv = _sc_compact(vol.reshape(-1))
    mask = maskv > 0

    value = jnp.concatenate(
        [jnp.transpose(f[0].reshape(C, -1), (1, 0)) for f in (feat0, feat1, feat2)], 0)
    NV = value.shape[0]

    v = pl.pallas_call(
        _vproj_body,
        out_shape=jax.ShapeDtypeStruct((NV, C), jnp.float32),
        grid=(NV // 384,),
        in_specs=[pl.BlockSpec((384, C), lambda i: (i, 0)),
                  pl.BlockSpec((C, C), lambda i: (0, 0)),
                  pl.BlockSpec((C,), lambda i: (0,))],
        out_specs=pl.BlockSpec((384, C), lambda i: (i, 0)),
    )(value, vp_w, vp_b)

    off, awl = pl.pallas_call(
        _pre_body,
        out_shape=(jax.ShapeDtypeStruct((N, NH * NL * NP * 2), jnp.float32),
                   jax.ShapeDtypeStruct((N, NH * NL * NP), jnp.float32)),
        grid=(N // BQ,),
        in_specs=[pl.BlockSpec((BQ, C), lambda i: (i, 0)),
                  pl.BlockSpec((BQ, C), lambda i: (i, 0)),
                  pl.BlockSpec((C, NH * NL * NP * 2), lambda i: (0, 0)),
                  pl.BlockSpec((NH * NL * NP * 2,), lambda i: (0,)),
                  pl.BlockSpec((C, NH * NL * NP), lambda i: (0, 0)),
                  pl.BlockSpec((NH * NL * NP,), lambda i: (0,))],
        out_specs=(pl.BlockSpec((BQ, NH * NL * NP * 2), lambda i: (i, 0)),
                   pl.BlockSpec((BQ, NH * NL * NP), lambda i: (i, 0))),
    )(q, qpos, so_w, so_b, aw_w, aw_b)

    vt = jnp.transpose(v.reshape(NV, NH, DH), (1, 0, 2)).reshape(NH * NV, DH)
    scale = jnp.array([64.0, 64.0, 32.0, 32.0, 16.0, 16.0], jnp.float32)
    refp6 = jnp.concatenate([ref[:, 0:1], ref[:, 1:2]] * 3, axis=1) * scale - 0.5
    refp = jnp.pad(refp6, ((0, 0), (0, 10)))

    sampled = _sc_sample(vt, off, awl, refp, idxc, cnt)

    out_final = pl.pallas_call(
        _post_body,
        out_shape=jax.ShapeDtypeStruct((N, C), jnp.float32),
        grid=(N // BQ,),
        in_specs=[pl.BlockSpec((BQ, DH), lambda i: (i, 0))] * NH
        + [pl.BlockSpec((BQ, C), lambda i: (i, 0)),
           pl.BlockSpec((BQ, 1), lambda i: (i, 0)),
           pl.BlockSpec((C, C), lambda i: (0, 0)),
           pl.BlockSpec((C,), lambda i: (0,)),
           pl.BlockSpec((C,), lambda i: (0,)),
           pl.BlockSpec((C,), lambda i: (0,)),
           pl.BlockSpec((C, 4 * C), lambda i: (0, 0)),
           pl.BlockSpec((4 * C,), lambda i: (0,)),
           pl.BlockSpec((4 * C, C), lambda i: (0, 0)),
           pl.BlockSpec((C,), lambda i: (0,)),
           pl.BlockSpec((C,), lambda i: (0,)),
           pl.BlockSpec((C,), lambda i: (0,))],
        out_specs=pl.BlockSpec((BQ, C), lambda i: (i, 0)),
    )(*[sampled[hh] for hh in range(NH)], q, maskv.reshape(N, 1),
      op_w, op_b, n1_g, n1_b, f1_w, f1_b, f2_w, f2_b, n2_g, n2_b)
    return out_final[None]


# speedup vs baseline: 1.2112x; 1.2112x over previous
import functools

import jax
import jax.numpy as jnp
from jax import lax
from jax.experimental import pallas as pl
from jax.experimental.pallas import tpu as pltpu
from jax.experimental.pallas import tpu_sc as plsc

SCENE = (64, 64, 8)
C = 128
NH = 8
NL = 3
NP = 4
DH = C // NH
LEVEL_SHAPES = [(64, 64), (32, 32), (16, 16)]

BQ = 256


def _pre_body(q_ref, qpos_ref, so_w_ref, so_b_ref, aw_w_ref, aw_b_ref,
              off_ref, awl_ref):
    qq = q_ref[...] + qpos_ref[...]
    off_ref[...] = jnp.dot(qq, so_w_ref[...],
                           preferred_element_type=jnp.float32) + so_b_ref[...]
    awl_ref[...] = jnp.dot(qq, aw_w_ref[...],
                           preferred_element_type=jnp.float32) + aw_b_ref[...]


def _vproj_body(val_ref, w_ref, b_ref, out_ref):
    out_ref[...] = jnp.dot(val_ref[...], w_ref[...],
                           preferred_element_type=jnp.float32) + b_ref[...]


def _post_body(s_ref, q_ref, mk_ref,
               op_w_ref, op_b_ref, n1g_ref, n1b_ref,
               f1w_ref, f1b_ref, f2w_ref, f2b_ref, n2g_ref, n2b_ref, out_ref):
    x = jnp.dot(s_ref[...], op_w_ref[...],
                preferred_element_type=jnp.float32) + op_b_ref[...] + q_ref[...]
    m = jnp.mean(x, -1, keepdims=True)
    v = jnp.mean((x - m) * (x - m), -1, keepdims=True)
    x = (x - m) * jax.lax.rsqrt(v + 1e-5) * n1g_ref[...] + n1b_ref[...]
    h = jnp.maximum(jnp.dot(x, f1w_ref[...],
                            preferred_element_type=jnp.float32) + f1b_ref[...], 0.0)
    y = x + jnp.dot(h, f2w_ref[...],
                    preferred_element_type=jnp.float32) + f2b_ref[...]
    m2 = jnp.mean(y, -1, keepdims=True)
    v2 = jnp.mean((y - m2) * (y - m2), -1, keepdims=True)
    pts = (y - m2) * jax.lax.rsqrt(v2 + 1e-5) * n2g_ref[...] + n2b_ref[...]
    out_ref[...] = jnp.where(mk_ref[...] > 0, pts, q_ref[...])


NC_SC = 2
NS_SC = 16
NW_SC = NC_SC * NS_SC
BQS = 64
NV_TOT = 64 * 64 + 32 * 32 + 16 * 16
LEVEL_STARTS = (0, 4096, 5120)


def _sc_sample_group(q0, tab_v, off_v, awl_v, refp_v, out_v, h):
    qloc = q0 + lax.broadcasted_iota(jnp.int32, (16,), 0)

    def col(c):
        return jnp.full((16,), c, jnp.int32)

    logits = [plsc.load_gather(awl_v, [qloc, col(h * 12 + lp)]) for lp in range(12)]
    m = logits[0]
    for t in logits[1:]:
        m = jnp.maximum(m, t)
    es = [jnp.exp(t - m) for t in logits]
    s = es[0]
    for t in es[1:]:
        s = s + t
    r = 1.0 / s
    aws = [e * r for e in es]

    accs = [jnp.zeros((16,), jnp.float32) for _ in range(DH)]
    for l, (Hl, Wl) in enumerate(LEVEL_SHAPES):
        rx = plsc.load_gather(refp_v, [qloc, col(2 * l)])
        ry = plsc.load_gather(refp_v, [qloc, col(2 * l + 1)])
        for p in range(NP):
            ocol = h * 24 + l * 8 + p * 2
            x = rx + plsc.load_gather(off_v, [qloc, col(ocol)])
            y = ry + plsc.load_gather(off_v, [qloc, col(ocol + 1)])
            tx = x.astype(jnp.int32).astype(jnp.float32)
            x0 = jnp.where(tx > x, tx - 1.0, tx)
            ty = y.astype(jnp.int32).astype(jnp.float32)
            y0 = jnp.where(ty > y, ty - 1.0, ty)
            fx = x - x0
            fy = y - y0
            aw = aws[l * NP + p]
            for dx in (0, 1):
                for dy in (0, 1):
                    xi = x0 + dx
                    yi = y0 + dy
                    valid = ((xi >= 0.0) & (xi <= Wl - 1.0)
                             & (yi >= 0.0) & (yi <= Hl - 1.0))
                    wc = (fx if dx else 1.0 - fx) * (fy if dy else 1.0 - fy)
                    wt = jnp.where(valid, wc * aw, 0.0)
                    xc = jnp.clip(xi, 0.0, Wl - 1.0).astype(jnp.int32)
                    yc = jnp.clip(yi, 0.0, Hl - 1.0).astype(jnp.int32)
                    row = (yc * Wl + xc + LEVEL_STARTS[l]) * DH
                    for d in range(DH):
                        g = plsc.load_gather(tab_v, [row + d])
                        accs[d] = accs[d] + wt * g
    for d in range(DH):
        plsc.store_scatter(out_v, [qloc, col(d)], accs[d])


NSEG = 16
SEGQ = SCENE[0] * SCENE[1] * SCENE[2] // NSEG


def _sc_compact_body(vol_hbm, idxc_hbm, cnt_hbm, mask_hbm,
                     loc_v, vol_v, win_v, tmp_v, idxl_v, mk_v, cnt16_v, table_s):
    X, Y, Z = SCENE
    N = X * Y * Z
    t = lax.axis_index("s")
    lane = lax.broadcasted_iota(jnp.int32, (16,), 0)

    def zbody(i, _):
        loc_v[pl.ds(i * 16, 16)] = jnp.zeros((16,), jnp.int32)
        return 0

    lax.fori_loop(0, N // 16, zbody, 0)
    pltpu.sync_copy(vol_hbm.at[pl.ds(t * (SEGQ * 3), SEGQ * 3)], vol_v)

    def sbody(g, _):
        r3 = (g * 16 + lane) * 3
        gx = plsc.load_gather(vol_v, [r3])
        gy = plsc.load_gather(vol_v, [r3 + 1])
        gz = plsc.load_gather(vol_v, [r3 + 2])
        keep = ((gx >= 0) & (gx < X) & (gy >= 0) & (gy < Y)
                & (gz >= 0) & (gz < Z))
        cx = jnp.clip(gx, 0, X - 1)
        cy = jnp.clip(gy, 0, Y - 1)
        cz = jnp.clip(gz, 0, Z - 1)
        f = (cx * Y + cy) * Z + cz
        plsc.store_scatter(loc_v, [f], jnp.ones((16,), jnp.int32), mask=keep)
        return 0

    lax.fori_loop(0, SEGQ // 16, sbody, 0)
    pltpu.sync_copy(loc_v, table_s.at[t])
    plsc.subcore_barrier()

    pltpu.sync_copy(table_s.at[0, pl.ds(t * SEGQ, SEGQ)], win_v)
    for j in range(1, NSEG):
        pltpu.sync_copy(table_s.at[j, pl.ds(t * SEGQ, SEGQ)], tmp_v)

        def abody(i, _):
            win_v[pl.ds(i * 16, 16)] = (win_v[pl.ds(i * 16, 16)]
                                        + tmp_v[pl.ds(i * 16, 16)])
            return 0

        lax.fori_loop(0, SEGQ // 16, abody, 0)

    def zbody2(i, _):
        idxl_v[pl.ds(i * 16, 16)] = jnp.zeros((16,), jnp.int32)
        return 0

    lax.fori_loop(0, (SEGQ + 64) // 16, zbody2, 0)

    def cbody(g, woff):
        cnt = win_v[pl.ds(g * 16, 16)]
        m = cnt > 0
        mk_v[pl.ds(g * 16, 16)] = m.astype(jnp.int32)
        pos = plsc.cumsum(m.astype(jnp.int32))
        widx = t * SEGQ + g * 16 + lane
        plsc.store_scatter(idxl_v, [woff + pos - 1], widx, mask=m)
        return woff + jnp.max(pos, axis=0)

    woff = lax.fori_loop(0, SEGQ // 16, cbody, jnp.int32(0))

    e0 = plsc.load_gather(idxl_v, [jnp.zeros((16,), jnp.int32)])
    pbase = (woff // 16) * 16
    rem = woff - pbase
    part = idxl_v[pl.ds(pbase, 16)]
    idxl_v[pl.ds(pbase, 16)] = jnp.where(lane < rem, part, e0)
    wpad = (woff + 63) // 64 * 64

    def pbody(i, _):
        idxl_v[pl.ds(pbase + 16 + i * 16, 16)] = e0
        return 0

    lax.fori_loop(0, (wpad - pbase - 16) // 16, pbody, 0)

    pltpu.sync_copy(idxl_v.at[pl.ds(0, SEGQ)], idxc_hbm.at[pl.ds(t * SEGQ, SEGQ)])
    pltpu.sync_copy(mk_v, mask_hbm.at[pl.ds(t * SEGQ, SEGQ)])
    cnt16_v[...] = jnp.broadcast_to(wpad // BQS, (16,)).astype(jnp.int32)
    pltpu.sync_copy(cnt16_v, cnt_hbm.at[pl.ds(t * 16, 16)])


def _sc_compact(vol_flat):
    N = SCENE[0] * SCENE[1] * SCENE[2]
    mesh = plsc.VectorSubcoreMesh(core_axis_name="c", subcore_axis_name="s",
                                  num_cores=1)
    f = pl.kernel(
        _sc_compact_body,
        mesh=mesh,
        out_type=(jax.ShapeDtypeStruct((N,), jnp.int32),
                  jax.ShapeDtypeStruct((NSEG * 16,), jnp.int32),
                  jax.ShapeDtypeStruct((N,), jnp.int32)),
        scratch_types=[
            pltpu.VMEM((N,), jnp.int32),
            pltpu.VMEM((SEGQ * 3,), jnp.int32),
            pltpu.VMEM((SEGQ,), jnp.int32),
            pltpu.VMEM((SEGQ,), jnp.int32),
            pltpu.VMEM((SEGQ + 64,), jnp.int32),
            pltpu.VMEM((SEGQ,), jnp.int32),
            pltpu.VMEM((16,), jnp.int32),
            pltpu.VMEM_SHARED((NSEG, N), jnp.int32),
        ],
        compiler_params=pltpu.CompilerParams(
            needs_layout_passes=False, use_tc_tiling_on_sc=False),
    )
    return f(vol_flat)


def _sc_sample_body(vt_hbm, off_hbm, awl_hbm, refp_hbm, idxc_hbm, cnt_hbm, out_hbm,
                    tab_v, off_v, awl_v, refp_v, out_v, idx_v, idx2_v, cnt_v):
    N = SCENE[0] * SCENE[1] * SCENE[2]
    w = lax.axis_index("s") * NC_SC + lax.axis_index("c")
    h = w % NH
    chunk = w // NH
    nchunks = NW_SC // NH

    pltpu.sync_copy(vt_hbm.at[pl.ds(h * (NV_TOT * DH), NV_TOT * DH)], tab_v)
    pltpu.sync_copy(cnt_hbm, cnt_v)
    lane = lax.broadcasted_iota(jnp.int32, (16,), 0)
    cnts = plsc.load_gather(cnt_v, [lane * 16])

    def bwork(base):
        pltpu.sync_copy(idxc_hbm.at[pl.ds(base, BQS)], idx_v)
        pltpu.sync_copy(off_hbm.at[idx_v], off_v)
        pltpu.sync_copy(awl_hbm.at[idx_v], awl_v)
        pltpu.sync_copy(refp_hbm.at[idx_v], refp_v)
        for gg in range(BQS // 16):
            idx2_v[pl.ds(gg * 16, 16)] = idx_v[pl.ds(gg * 16, 16)] + h * N

        def qbody(qg, _):
            _sc_sample_group(qg * 16, tab_v, off_v, awl_v, refp_v, out_v, h)
            return 0

        lax.fori_loop(0, BQS // 16, qbody, 0)
        pltpu.sync_copy(out_v, out_hbm.at[idx2_v])

    def seg_body(s, _):
        nbs = jnp.sum(jnp.where(lane == s, cnts, 0), axis=0)
        nbc = (nbs - chunk + 3) // 4

        def bbody(bi, _):
            bwork(s * SEGQ + (bi * nchunks + chunk) * BQS)
            return 0

        lax.fori_loop(0, nbc, bbody, 0)
        return 0

    lax.fori_loop(0, NSEG, seg_body, 0)


def _sc_sample(vt, off, awl, refp, idxc, cnt):
    N = SCENE[0] * SCENE[1] * SCENE[2]
    mesh = plsc.VectorSubcoreMesh(core_axis_name="c", subcore_axis_name="s")
    f = pl.kernel(
        _sc_sample_body,
        mesh=mesh,
        out_type=jax.ShapeDtypeStruct((NH * N, DH), jnp.float32),
        scratch_types=[
            pltpu.VMEM((NV_TOT * DH,), jnp.float32),
            pltpu.VMEM((BQS, NH * NL * NP * 2), jnp.float32),
            pltpu.VMEM((BQS, NH * NL * NP), jnp.float32),
            pltpu.VMEM((BQS, 16), jnp.float32),
            pltpu.VMEM((BQS, DH), jnp.float32),
            pltpu.VMEM((BQS,), jnp.int32),
            pltpu.VMEM((BQS,), jnp.int32),
            pltpu.VMEM((NSEG * 16,), jnp.int32),
        ],
        compiler_params=pltpu.CompilerParams(
            needs_layout_passes=False, use_tc_tiling_on_sc=False),
    )
    out = f(vt.reshape(-1), off, awl, refp, idxc, cnt)
    return jnp.transpose(out.reshape(NH, N, DH), (1, 0, 2)).reshape(N, C)


def _bilinear_all(vh, x, y, Hl, Wl):
    x0 = jnp.floor(x)
    y0 = jnp.floor(y)

    def g(xi, yi):
        valid = (xi >= 0) & (xi <= Wl - 1) & (yi >= 0) & (yi <= Hl - 1)
        xc = jnp.clip(xi, 0, Wl - 1).astype(jnp.int32)
        yc = jnp.clip(yi, 0, Hl - 1).astype(jnp.int32)
        idx = yc * Wl + xc
        got = jnp.take_along_axis(vh, idx.reshape(NH, -1, 1), axis=1)
        return got.reshape(xi.shape + (DH,)) * valid[..., None]

    w00 = (x0 + 1 - x) * (y0 + 1 - y)
    w01 = (x - x0) * (y0 + 1 - y)
    w10 = (x0 + 1 - x) * (y - y0)
    w11 = (x - x0) * (y - y0)
    return (g(x0, y0) * w00[..., None] + g(x0 + 1, y0) * w01[..., None]
            + g(x0, y0 + 1) * w10[..., None] + g(x0 + 1, y0 + 1) * w11[..., None])


def kernel(scene_embed, feat0, feat1, feat2, scene_pos, ref_pix, vol_pts,
           vp_w, vp_b, so_w, so_b, aw_w, aw_b, op_w, op_b,
           n1_g, n1_b, f1_w, f1_b, f2_w, f2_b, n2_g, n2_b):
    X, Y, Z = SCENE
    N = X * Y * Z
    q = scene_embed[0]
    qpos = scene_pos[0]
    vol = vol_pts[0]
    ref = ref_pix[0]

    idxc, cnt, maskv = _sc_compact(vol.reshape(-1))
    mask = maskv > 0

    value = jnp.concatenate(
        [jnp.transpose(f[0].reshape(C, -1), (1, 0)) for f in (feat0, feat1, feat2)], 0)
    NV = value.shape[0]

    v = pl.pallas_call(
        _vproj_body,
        out_shape=jax.ShapeDtypeStruct((NV, C), jnp.float32),
        grid=(NV // 384,),
        in_specs=[pl.BlockSpec((384, C), lambda i: (i, 0)),
                  pl.BlockSpec((C, C), lambda i: (0, 0)),
                  pl.BlockSpec((C,), lambda i: (0,))],
        out_specs=pl.BlockSpec((384, C), lambda i: (i, 0)),
    )(value, vp_w, vp_b)

    off, awl = pl.pallas_call(
        _pre_body,
        out_shape=(jax.ShapeDtypeStruct((N, NH * NL * NP * 2), jnp.float32),
                   jax.ShapeDtypeStruct((N, NH * NL * NP), jnp.float32)),
        grid=(N // BQ,),
        in_specs=[pl.BlockSpec((BQ, C), lambda i: (i, 0)),
                  pl.BlockSpec((BQ, C), lambda i: (i, 0)),
                  pl.BlockSpec((C, NH * NL * NP * 2), lambda i: (0, 0)),
                  pl.BlockSpec((NH * NL * NP * 2,), lambda i: (0,)),
                  pl.BlockSpec((C, NH * NL * NP), lambda i: (0, 0)),
                  pl.BlockSpec((NH * NL * NP,), lambda i: (0,))],
        out_specs=(pl.BlockSpec((BQ, NH * NL * NP * 2), lambda i: (i, 0)),
                   pl.BlockSpec((BQ, NH * NL * NP), lambda i: (i, 0))),
    )(q, qpos, so_w, so_b, aw_w, aw_b)

    vt = jnp.transpose(v.reshape(NV, NH, DH), (1, 0, 2)).reshape(NH * NV, DH)
    scale = jnp.array([64.0, 64.0, 32.0, 32.0, 16.0, 16.0], jnp.float32)
    refp6 = jnp.concatenate([ref[:, 0:1], ref[:, 1:2]] * 3, axis=1) * scale - 0.5
    refp = jnp.pad(refp6, ((0, 0), (0, 10)))

    sampled = _sc_sample(vt, off, awl, refp, idxc, cnt)

    out_final = pl.pallas_call(
        _post_body,
        out_shape=jax.ShapeDtypeStruct((N, C), jnp.float32),
        grid=(N // BQ,),
        in_specs=[pl.BlockSpec((BQ, C), lambda i: (i, 0)),
           pl.BlockSpec((BQ, C), lambda i: (i, 0)),
           pl.BlockSpec((BQ, 1), lambda i: (i, 0)),
           pl.BlockSpec((C, C), lambda i: (0, 0)),
           pl.BlockSpec((C,), lambda i: (0,)),
           pl.BlockSpec((C,), lambda i: (0,)),
           pl.BlockSpec((C,), lambda i: (0,)),
           pl.BlockSpec((C, 4 * C), lambda i: (0, 0)),
           pl.BlockSpec((4 * C,), lambda i: (0,)),
           pl.BlockSpec((4 * C, C), lambda i: (0, 0)),
           pl.BlockSpec((C,), lambda i: (0,)),
           pl.BlockSpec((C,), lambda i: (0,)),
           pl.BlockSpec((C,), lambda i: (0,))],
        out_specs=pl.BlockSpec((BQ, C), lambda i: (i, 0)),
    )(sampled, q, maskv.reshape(N, 1),
      op_w, op_b, n1_g, n1_b, f1_w, f1_b, f2_w, f2_b, n2_g, n2_b)
    return out_final[None]


# speedup vs baseline: 1.3677x; 1.1292x over previous
import functools

import jax
import jax.numpy as jnp
from jax import lax
from jax.experimental import pallas as pl
from jax.experimental.pallas import tpu as pltpu
from jax.experimental.pallas import tpu_sc as plsc

SCENE = (64, 64, 8)
C = 128
NH = 8
NL = 3
NP = 4
DH = C // NH
LEVEL_SHAPES = [(64, 64), (32, 32), (16, 16)]

BQ = 512


def _pre_body(q_ref, qpos_ref, so_w_ref, so_b_ref, aw_w_ref, aw_b_ref,
              off_ref, awl_ref):
    qq = q_ref[...] + qpos_ref[...]
    off_ref[...] = jnp.dot(qq, so_w_ref[...],
                           preferred_element_type=jnp.float32) + so_b_ref[...]
    awl_ref[...] = jnp.dot(qq, aw_w_ref[...],
                           preferred_element_type=jnp.float32) + aw_b_ref[...]


def _vproj_body(val_ref, w_ref, b_ref, out_ref):
    out_ref[...] = jnp.dot(val_ref[...], w_ref[...],
                           preferred_element_type=jnp.float32) + b_ref[...]


def _post_body(s_ref, q_ref, mk_ref,
               op_w_ref, op_b_ref, n1g_ref, n1b_ref,
               f1w_ref, f1b_ref, f2w_ref, f2b_ref, n2g_ref, n2b_ref, out_ref):
    x = jnp.dot(s_ref[...], op_w_ref[...],
                preferred_element_type=jnp.float32) + op_b_ref[...] + q_ref[...]
    m = jnp.mean(x, -1, keepdims=True)
    v = jnp.mean((x - m) * (x - m), -1, keepdims=True)
    x = (x - m) * jax.lax.rsqrt(v + 1e-5) * n1g_ref[...] + n1b_ref[...]
    h = jnp.maximum(jnp.dot(x, f1w_ref[...],
                            preferred_element_type=jnp.float32) + f1b_ref[...], 0.0)
    y = x + jnp.dot(h, f2w_ref[...],
                    preferred_element_type=jnp.float32) + f2b_ref[...]
    m2 = jnp.mean(y, -1, keepdims=True)
    v2 = jnp.mean((y - m2) * (y - m2), -1, keepdims=True)
    pts = (y - m2) * jax.lax.rsqrt(v2 + 1e-5) * n2g_ref[...] + n2b_ref[...]
    out_ref[...] = jnp.where(mk_ref[...] > 0, pts, q_ref[...])


NC_SC = 2
NS_SC = 16
NW_SC = NC_SC * NS_SC
BQS = 64
NV_TOT = 64 * 64 + 32 * 32 + 16 * 16
LEVEL_STARTS = (0, 4096, 5120)


def _sc_sample_group(q0, tab_v, off_v, awl_v, refp_v, out_v, h):
    qloc = q0 + lax.broadcasted_iota(jnp.int32, (16,), 0)

    def col(c):
        return jnp.full((16,), c, jnp.int32)

    logits = [plsc.load_gather(awl_v, [qloc, col(h * 12 + lp)]) for lp in range(12)]
    m = logits[0]
    for t in logits[1:]:
        m = jnp.maximum(m, t)
    es = [jnp.exp(t - m) for t in logits]
    s = es[0]
    for t in es[1:]:
        s = s + t
    r = 1.0 / s
    aws = [e * r for e in es]

    accs = [jnp.zeros((16,), jnp.float32) for _ in range(DH)]
    for l, (Hl, Wl) in enumerate(LEVEL_SHAPES):
        rx = plsc.load_gather(refp_v, [qloc, col(2 * l)])
        ry = plsc.load_gather(refp_v, [qloc, col(2 * l + 1)])
        for p in range(NP):
            ocol = h * 24 + l * 8 + p * 2
            x = rx + plsc.load_gather(off_v, [qloc, col(ocol)])
            y = ry + plsc.load_gather(off_v, [qloc, col(ocol + 1)])
            tx = x.astype(jnp.int32).astype(jnp.float32)
            x0 = jnp.where(tx > x, tx - 1.0, tx)
            ty = y.astype(jnp.int32).astype(jnp.float32)
            y0 = jnp.where(ty > y, ty - 1.0, ty)
            fx = x - x0
            fy = y - y0
            aw = aws[l * NP + p]
            for dx in (0, 1):
                for dy in (0, 1):
                    xi = x0 + dx
                    yi = y0 + dy
                    valid = ((xi >= 0.0) & (xi <= Wl - 1.0)
                             & (yi >= 0.0) & (yi <= Hl - 1.0))
                    wc = (fx if dx else 1.0 - fx) * (fy if dy else 1.0 - fy)
                    wt = jnp.where(valid, wc * aw, 0.0)
                    xc = jnp.clip(xi, 0.0, Wl - 1.0).astype(jnp.int32)
                    yc = jnp.clip(yi, 0.0, Hl - 1.0).astype(jnp.int32)
                    row = (yc * Wl + xc + LEVEL_STARTS[l]) * DH
                    for d in range(DH):
                        g = plsc.load_gather(tab_v, [row + d])
                        accs[d] = accs[d] + wt * g
    for d in range(DH):
        plsc.store_scatter(out_v, [qloc, col(d)], accs[d])


NSEG = 16
SEGQ = SCENE[0] * SCENE[1] * SCENE[2] // NSEG


def _sc_compact_body(vol_hbm, idxc_hbm, cnt_hbm, mask_hbm,
                     loc_v, vol_v, win_v, tmp_v, idxl_v, mk_v, cnt16_v, table_s):
    X, Y, Z = SCENE
    N = X * Y * Z
    t = lax.axis_index("s")
    lane = lax.broadcasted_iota(jnp.int32, (16,), 0)

    def zbody(i, _):
        loc_v[pl.ds(i * 16, 16)] = jnp.zeros((16,), jnp.int32)
        return 0

    lax.fori_loop(0, N // 16, zbody, 0)
    pltpu.sync_copy(vol_hbm.at[pl.ds(t * (SEGQ * 3), SEGQ * 3)], vol_v)

    def sbody(g, _):
        r3 = (g * 16 + lane) * 3
        gx = plsc.load_gather(vol_v, [r3])
        gy = plsc.load_gather(vol_v, [r3 + 1])
        gz = plsc.load_gather(vol_v, [r3 + 2])
        keep = ((gx >= 0) & (gx < X) & (gy >= 0) & (gy < Y)
                & (gz >= 0) & (gz < Z))
        cx = jnp.clip(gx, 0, X - 1)
        cy = jnp.clip(gy, 0, Y - 1)
        cz = jnp.clip(gz, 0, Z - 1)
        f = (cx * Y + cy) * Z + cz
        plsc.store_scatter(loc_v, [f], jnp.ones((16,), jnp.int32), mask=keep)
        return 0

    lax.fori_loop(0, SEGQ // 16, sbody, 0)
    pltpu.sync_copy(loc_v, table_s.at[t])
    plsc.subcore_barrier()

    pltpu.sync_copy(table_s.at[0, pl.ds(t * SEGQ, SEGQ)], win_v)
    for j in range(1, NSEG):
        pltpu.sync_copy(table_s.at[j, pl.ds(t * SEGQ, SEGQ)], tmp_v)

        def abody(i, _):
            win_v[pl.ds(i * 16, 16)] = (win_v[pl.ds(i * 16, 16)]
                                        + tmp_v[pl.ds(i * 16, 16)])
            return 0

        lax.fori_loop(0, SEGQ // 16, abody, 0)

    def zbody2(i, _):
        idxl_v[pl.ds(i * 16, 16)] = jnp.zeros((16,), jnp.int32)
        return 0

    lax.fori_loop(0, (SEGQ + 64) // 16, zbody2, 0)

    def cbody(g, woff):
        cnt = win_v[pl.ds(g * 16, 16)]
        m = cnt > 0
        mk_v[pl.ds(g * 16, 16)] = m.astype(jnp.int32)
        pos = plsc.cumsum(m.astype(jnp.int32))
        widx = t * SEGQ + g * 16 + lane
        plsc.store_scatter(idxl_v, [woff + pos - 1], widx, mask=m)
        return woff + jnp.max(pos, axis=0)

    woff = lax.fori_loop(0, SEGQ // 16, cbody, jnp.int32(0))

    e0 = plsc.load_gather(idxl_v, [jnp.zeros((16,), jnp.int32)])
    pbase = (woff // 16) * 16
    rem = woff - pbase
    part = idxl_v[pl.ds(pbase, 16)]
    idxl_v[pl.ds(pbase, 16)] = jnp.where(lane < rem, part, e0)
    wpad = (woff + 63) // 64 * 64

    def pbody(i, _):
        idxl_v[pl.ds(pbase + 16 + i * 16, 16)] = e0
        return 0

    lax.fori_loop(0, (wpad - pbase - 16) // 16, pbody, 0)

    pltpu.sync_copy(idxl_v.at[pl.ds(0, SEGQ)], idxc_hbm.at[pl.ds(t * SEGQ, SEGQ)])
    pltpu.sync_copy(mk_v, mask_hbm.at[pl.ds(t * SEGQ, SEGQ)])
    cnt16_v[...] = jnp.broadcast_to(wpad // BQS, (16,)).astype(jnp.int32)
    pltpu.sync_copy(cnt16_v, cnt_hbm.at[pl.ds(t * 16, 16)])


def _sc_compact(vol_flat):
    N = SCENE[0] * SCENE[1] * SCENE[2]
    mesh = plsc.VectorSubcoreMesh(core_axis_name="c", subcore_axis_name="s",
                                  num_cores=1)
    f = pl.kernel(
        _sc_compact_body,
        mesh=mesh,
        out_type=(jax.ShapeDtypeStruct((N,), jnp.int32),
                  jax.ShapeDtypeStruct((NSEG * 16,), jnp.int32),
                  jax.ShapeDtypeStruct((N,), jnp.int32)),
        scratch_types=[
            pltpu.VMEM((N,), jnp.int32),
            pltpu.VMEM((SEGQ * 3,), jnp.int32),
            pltpu.VMEM((SEGQ,), jnp.int32),
            pltpu.VMEM((SEGQ,), jnp.int32),
            pltpu.VMEM((SEGQ + 64,), jnp.int32),
            pltpu.VMEM((SEGQ,), jnp.int32),
            pltpu.VMEM((16,), jnp.int32),
            pltpu.VMEM_SHARED((NSEG, N), jnp.int32),
        ],
        compiler_params=pltpu.CompilerParams(
            needs_layout_passes=False, use_tc_tiling_on_sc=False),
    )
    return f(vol_flat)


def _sc_sample_body(vt_hbm, off_hbm, awl_hbm, refp_hbm, idxc_hbm, cnt_hbm, out_hbm,
                    tab_v, off_v, awl_v, refp_v, out_v, idx_v, idx2_v, cnt_v):
    N = SCENE[0] * SCENE[1] * SCENE[2]
    w = lax.axis_index("s") * NC_SC + lax.axis_index("c")
    h = w % NH
    chunk = w // NH
    nchunks = NW_SC // NH

    pltpu.sync_copy(vt_hbm.at[pl.ds(h * (NV_TOT * DH), NV_TOT * DH)], tab_v)
    pltpu.sync_copy(cnt_hbm, cnt_v)
    lane = lax.broadcasted_iota(jnp.int32, (16,), 0)
    cnts = plsc.load_gather(cnt_v, [lane * 16])

    def bwork(base):
        pltpu.sync_copy(idxc_hbm.at[pl.ds(base, BQS)], idx_v)
        pltpu.sync_copy(off_hbm.at[idx_v], off_v)
        pltpu.sync_copy(awl_hbm.at[idx_v], awl_v)
        pltpu.sync_copy(refp_hbm.at[idx_v], refp_v)
        for gg in range(BQS // 16):
            idx2_v[pl.ds(gg * 16, 16)] = idx_v[pl.ds(gg * 16, 16)] + h * N

        def qbody(qg, _):
            _sc_sample_group(qg * 16, tab_v, off_v, awl_v, refp_v, out_v, h)
            return 0

        lax.fori_loop(0, BQS // 16, qbody, 0)
        pltpu.sync_copy(out_v, out_hbm.at[idx2_v])

    def seg_body(s, _):
        nbs = jnp.sum(jnp.where(lane == s, cnts, 0), axis=0)
        nbc = (nbs - chunk + 3) // 4

        def bbody(bi, _):
            bwork(s * SEGQ + (bi * nchunks + chunk) * BQS)
            return 0

        lax.fori_loop(0, nbc, bbody, 0)
        return 0

    lax.fori_loop(0, NSEG, seg_body, 0)


def _sc_sample(vt, off, awl, refp, idxc, cnt):
    N = SCENE[0] * SCENE[1] * SCENE[2]
    mesh = plsc.VectorSubcoreMesh(core_axis_name="c", subcore_axis_name="s")
    f = pl.kernel(
        _sc_sample_body,
        mesh=mesh,
        out_type=jax.ShapeDtypeStruct((NH * N, DH), jnp.float32),
        scratch_types=[
            pltpu.VMEM((NV_TOT * DH,), jnp.float32),
            pltpu.VMEM((BQS, NH * NL * NP * 2), jnp.float32),
            pltpu.VMEM((BQS, NH * NL * NP), jnp.float32),
            pltpu.VMEM((BQS, 16), jnp.float32),
            pltpu.VMEM((BQS, DH), jnp.float32),
            pltpu.VMEM((BQS,), jnp.int32),
            pltpu.VMEM((BQS,), jnp.int32),
            pltpu.VMEM((NSEG * 16,), jnp.int32),
        ],
        compiler_params=pltpu.CompilerParams(
            needs_layout_passes=False, use_tc_tiling_on_sc=False),
    )
    out = f(vt.reshape(-1), off, awl, refp, idxc, cnt)
    return jnp.transpose(out.reshape(NH, N, DH), (1, 0, 2)).reshape(N, C)


def _bilinear_all(vh, x, y, Hl, Wl):
    x0 = jnp.floor(x)
    y0 = jnp.floor(y)

    def g(xi, yi):
        valid = (xi >= 0) & (xi <= Wl - 1) & (yi >= 0) & (yi <= Hl - 1)
        xc = jnp.clip(xi, 0, Wl - 1).astype(jnp.int32)
        yc = jnp.clip(yi, 0, Hl - 1).astype(jnp.int32)
        idx = yc * Wl + xc
        got = jnp.take_along_axis(vh, idx.reshape(NH, -1, 1), axis=1)
        return got.reshape(xi.shape + (DH,)) * valid[..., None]

    w00 = (x0 + 1 - x) * (y0 + 1 - y)
    w01 = (x - x0) * (y0 + 1 - y)
    w10 = (x0 + 1 - x) * (y - y0)
    w11 = (x - x0) * (y - y0)
    return (g(x0, y0) * w00[..., None] + g(x0 + 1, y0) * w01[..., None]
            + g(x0, y0 + 1) * w10[..., None] + g(x0 + 1, y0 + 1) * w11[..., None])


def kernel(scene_embed, feat0, feat1, feat2, scene_pos, ref_pix, vol_pts,
           vp_w, vp_b, so_w, so_b, aw_w, aw_b, op_w, op_b,
           n1_g, n1_b, f1_w, f1_b, f2_w, f2_b, n2_g, n2_b):
    X, Y, Z = SCENE
    N = X * Y * Z
    q = scene_embed[0]
    qpos = scene_pos[0]
    vol = vol_pts[0]
    ref = ref_pix[0]

    idxc, cnt, maskv = _sc_compact(vol.reshape(-1))
    mask = maskv > 0

    value = jnp.concatenate(
        [jnp.transpose(f[0].reshape(C, -1), (1, 0)) for f in (feat0, feat1, feat2)], 0)
    NV = value.shape[0]

    v = pl.pallas_call(
        _vproj_body,
        out_shape=jax.ShapeDtypeStruct((NV, C), jnp.float32),
        grid=(NV // 384,),
        in_specs=[pl.BlockSpec((384, C), lambda i: (i, 0)),
                  pl.BlockSpec((C, C), lambda i: (0, 0)),
                  pl.BlockSpec((C,), lambda i: (0,))],
        out_specs=pl.BlockSpec((384, C), lambda i: (i, 0)),
    )(value, vp_w, vp_b)

    off, awl = pl.pallas_call(
        _pre_body,
        out_shape=(jax.ShapeDtypeStruct((N, NH * NL * NP * 2), jnp.float32),
                   jax.ShapeDtypeStruct((N, NH * NL * NP), jnp.float32)),
        grid=(N // BQ,),
        in_specs=[pl.BlockSpec((BQ, C), lambda i: (i, 0)),
                  pl.BlockSpec((BQ, C), lambda i: (i, 0)),
                  pl.BlockSpec((C, NH * NL * NP * 2), lambda i: (0, 0)),
                  pl.BlockSpec((NH * NL * NP * 2,), lambda i: (0,)),
                  pl.BlockSpec((C, NH * NL * NP), lambda i: (0, 0)),
                  pl.BlockSpec((NH * NL * NP,), lambda i: (0,))],
        out_specs=(pl.BlockSpec((BQ, NH * NL * NP * 2), lambda i: (i, 0)),
                   pl.BlockSpec((BQ, NH * NL * NP), lambda i: (i, 0))),
    )(q, qpos, so_w, so_b, aw_w, aw_b)

    vt = jnp.transpose(v.reshape(NV, NH, DH), (1, 0, 2)).reshape(NH * NV, DH)
    scale = jnp.array([64.0, 64.0, 32.0, 32.0, 16.0, 16.0], jnp.float32)
    refp6 = jnp.concatenate([ref[:, 0:1], ref[:, 1:2]] * 3, axis=1) * scale - 0.5
    refp = jnp.pad(refp6, ((0, 0), (0, 10)))

    sampled = _sc_sample(vt, off, awl, refp, idxc, cnt)

    out_final = pl.pallas_call(
        _post_body,
        out_shape=jax.ShapeDtypeStruct((N, C), jnp.float32),
        grid=(N // BQ,),
        in_specs=[pl.BlockSpec((BQ, C), lambda i: (i, 0)),
           pl.BlockSpec((BQ, C), lambda i: (i, 0)),
           pl.BlockSpec((BQ, 1), lambda i: (i, 0)),
           pl.BlockSpec((C, C), lambda i: (0, 0)),
           pl.BlockSpec((C,), lambda i: (0,)),
           pl.BlockSpec((C,), lambda i: (0,)),
           pl.BlockSpec((C,), lambda i: (0,)),
           pl.BlockSpec((C, 4 * C), lambda i: (0, 0)),
           pl.BlockSpec((4 * C,), lambda i: (0,)),
           pl.BlockSpec((4 * C, C), lambda i: (0, 0)),
           pl.BlockSpec((C,), lambda i: (0,)),
           pl.BlockSpec((C,), lambda i: (0,)),
           pl.BlockSpec((C,), lambda i: (0,))],
        out_specs=pl.BlockSpec((BQ, C), lambda i: (i, 0)),
    )(sampled, q, maskv.reshape(N, 1),
      op_w, op_b, n1_g, n1_b, f1_w, f1_b, f2_w, f2_b, n2_g, n2_b)
    return out_final[None]
